# Initial kernel scaffold; baseline (speedup 1.0000x reference)
import jax
import jax.numpy as jnp
from jax.experimental import pallas as pl


def kernel(x, edge_index, W1, b1, W2, b2):
    raise NotImplementedError("write your pallas kernel here")


# speedup vs baseline: 10.4453x; 10.4453x over previous
import functools

import jax
import jax.numpy as jnp
from jax import lax
from jax.experimental import pallas as pl
from jax.experimental.pallas import tpu as pltpu
from jax.experimental.pallas import tpu_sc as plsc

NC = 2
NS = 16
L = 16
NW = NC * NS
CHUNK = 128
ZR = 64


def _round_up(v, m):
    return (v + m - 1) // m * m


def _mesh():
    return plsc.VectorSubcoreMesh(
        core_axis_name="c", subcore_axis_name="s",
        num_cores=NC, num_subcores=NS)


def _make_deg(npad, epad):
    ept = epad // NW
    nch = ept // CHUNK
    npr = npad // NS

    def body(dst_hbm, ones_hbm, zeros_hbm, deg_hbm, dchunk, onesb, deg_sh):
        cid = lax.axis_index("c")
        sid = lax.axis_index("s")
        gid = cid * NS + sid

        pltpu.sync_copy(ones_hbm, onesb)
        pltpu.sync_copy(zeros_hbm, deg_sh.at[pl.ds(sid * npr, npr)])
        plsc.subcore_barrier()

        def step(t, _):
            base = gid * ept + t * CHUNK
            pltpu.sync_copy(dst_hbm.at[pl.ds(base, CHUNK)], dchunk)
            pltpu.sync_copy(onesb, deg_sh.at[dchunk], add=True)
            return 0
        lax.fori_loop(0, nch, step, 0)
        plsc.subcore_barrier()

        pltpu.sync_copy(deg_sh.at[pl.ds(sid * npr, npr)],
                        deg_hbm.at[cid, pl.ds(sid * npr, npr)])

    return pl.kernel(
        body,
        out_type=jax.ShapeDtypeStruct((NC, npad), jnp.float32),
        mesh=_mesh(),
        scratch_types=[
            pltpu.VMEM((CHUNK,), jnp.int32),
            pltpu.VMEM((CHUNK,), jnp.float32),
            pltpu.VMEM_SHARED((npad,), jnp.float32),
        ],
    )


def _make_agg(npad, d, epad):
    ept = epad // NW
    nch = ept // CHUNK
    npr = npad // NS

    def body(y_hbm, src_hbm, dst_hbm, out_hbm, sidx, didx, rows, zb, acc_sh,
             sem):
        cid = lax.axis_index("c")
        sid = lax.axis_index("s")
        gid = cid * NS + sid

        zero = jnp.zeros((L,), jnp.float32)

        def zrow(r, _):
            for c in range(d // L):
                zb[r, pl.ds(c * L, L)] = zero
            return 0
        lax.fori_loop(0, ZR, zrow, 0)

        def zacc(k, _):
            pltpu.sync_copy(zb, acc_sh.at[pl.ds(sid * npr + k * ZR, ZR)])
            return 0
        lax.fori_loop(0, npr // ZR, zacc, 0)
        plsc.subcore_barrier()

        def step(t, _):
            base = gid * ept + t * CHUNK
            pltpu.sync_copy(src_hbm.at[pl.ds(base, CHUNK)], sidx)
            pltpu.sync_copy(dst_hbm.at[pl.ds(base, CHUNK)], didx)
            pltpu.async_copy(y_hbm.at[sidx], rows, sem).wait()
            pltpu.sync_copy(rows, acc_sh.at[didx], add=True)
            return 0
        lax.fori_loop(0, nch, step, 0)
        plsc.subcore_barrier()

        pltpu.sync_copy(
            acc_sh.at[pl.ds(sid * npr, npr)],
            out_hbm.at[cid, pl.ds(sid * npr, npr)])

    return pl.kernel(
        body,
        out_type=jax.ShapeDtypeStruct((NC, npad, d), jnp.float32),
        mesh=_mesh(),
        scratch_types=[
            pltpu.VMEM((CHUNK,), jnp.int32),
            pltpu.VMEM((CHUNK,), jnp.int32),
            pltpu.VMEM((CHUNK, d), jnp.float32),
            pltpu.VMEM((ZR, d), jnp.float32),
            pltpu.VMEM_SHARED((npad, d), jnp.float32),
            pltpu.SemaphoreType.DMA,
        ],
    )


def _block_rows(n):
    for b in (1024, 1000, 800, 640, 512, 400, 256, 200, 128, 100, 80, 64, 40,
              16, 8):
        if n % b == 0:
            return b
    return n


def _mm_call(n, d, br):
    def body(x_ref, w_ref, o_ref):
        o_ref[...] = jnp.dot(x_ref[...], w_ref[...],
                             preferred_element_type=jnp.float32)
    return pl.pallas_call(
        body,
        grid=(n // br,),
        in_specs=[pl.BlockSpec((br, d), lambda i: (i, 0)),
                  pl.BlockSpec((d, d), lambda i: (0, 0))],
        out_specs=pl.BlockSpec((br, d), lambda i: (i, 0)),
        out_shape=jax.ShapeDtypeStruct((n, d), jnp.float32),
    )


def _scale_call(n, d, br):
    def body(xw_ref, d0_ref, d1_ref, o_ref):
        dinv = lax.rsqrt(d0_ref[...] + d1_ref[...] + 1.0)
        o_ref[...] = xw_ref[...] * dinv
    return pl.pallas_call(
        body,
        grid=(n // br,),
        in_specs=[pl.BlockSpec((br, d), lambda i: (i, 0)),
                  pl.BlockSpec((br, 1), lambda i: (i, 0)),
                  pl.BlockSpec((br, 1), lambda i: (i, 0))],
        out_specs=pl.BlockSpec((br, d), lambda i: (i, 0)),
        out_shape=jax.ShapeDtypeStruct((n, d), jnp.float32),
    )


def _layer2_call(n, d, br):
    def body(p0_ref, p1_ref, y1_ref, d0_ref, d1_ref, b_ref, w_ref, o_ref):
        dinv = lax.rsqrt(d0_ref[...] + d1_ref[...] + 1.0)
        h = dinv * (p0_ref[...] + p1_ref[...] + y1_ref[...]) + b_ref[...]
        h = jnp.maximum(h, 0.0)
        o_ref[...] = dinv * jnp.dot(h, w_ref[...],
                                    preferred_element_type=jnp.float32)
    return pl.pallas_call(
        body,
        grid=(n // br,),
        in_specs=[pl.BlockSpec((br, d), lambda i: (i, 0)),
                  pl.BlockSpec((br, d), lambda i: (i, 0)),
                  pl.BlockSpec((br, d), lambda i: (i, 0)),
                  pl.BlockSpec((br, 1), lambda i: (i, 0)),
                  pl.BlockSpec((br, 1), lambda i: (i, 0)),
                  pl.BlockSpec((1, d), lambda i: (0, 0)),
                  pl.BlockSpec((d, d), lambda i: (0, 0))],
        out_specs=pl.BlockSpec((br, d), lambda i: (i, 0)),
        out_shape=jax.ShapeDtypeStruct((n, d), jnp.float32),
    )


def _final_call(n, d, br):
    def body(q0_ref, q1_ref, y2_ref, d0_ref, d1_ref, b_ref, o_ref):
        dinv = lax.rsqrt(d0_ref[...] + d1_ref[...] + 1.0)
        o_ref[...] = dinv * (q0_ref[...] + q1_ref[...] + y2_ref[...]) \
            + b_ref[...]
    return pl.pallas_call(
        body,
        grid=(n // br,),
        in_specs=[pl.BlockSpec((br, d), lambda i: (i, 0)),
                  pl.BlockSpec((br, d), lambda i: (i, 0)),
                  pl.BlockSpec((br, d), lambda i: (i, 0)),
                  pl.BlockSpec((br, 1), lambda i: (i, 0)),
                  pl.BlockSpec((br, 1), lambda i: (i, 0)),
                  pl.BlockSpec((1, d), lambda i: (0, 0))],
        out_specs=pl.BlockSpec((br, d), lambda i: (i, 0)),
        out_shape=jax.ShapeDtypeStruct((n, d), jnp.float32),
    )


def kernel(x, edge_index, W1, b1, W2, b2):
    n, d = x.shape
    e = edge_index.shape[1]
    npad = _round_up(n + 1, NS * ZR)
    epad = _round_up(e, NW * CHUNK)
    br = _block_rows(n)

    pad = epad - e
    srcp = jnp.concatenate(
        [edge_index[0], jnp.zeros((pad,), edge_index.dtype)])
    dstp = jnp.concatenate(
        [edge_index[1], jnp.full((pad,), n, edge_index.dtype)])

    ones_blk = jnp.ones((CHUNK,), jnp.float32)
    zeros_blk = jnp.zeros((npad // NS,), jnp.float32)
    deg = _make_deg(npad, epad)(dstp, ones_blk, zeros_blk)
    d0 = deg[0, :n].reshape(n, 1)
    d1 = deg[1, :n].reshape(n, 1)

    agg = _make_agg(npad, d, epad)
    mm = _mm_call(n, d, br)
    scale = _scale_call(n, d, br)
    layer2 = _layer2_call(n, d, br)
    final = _final_call(n, d, br)

    xw1 = mm(x, W1)
    y1 = scale(xw1, d0, d1)
    a1 = agg(y1, srcp, dstp)
    y2 = layer2(a1[0, :n], a1[1, :n], y1, d0, d1, b1.reshape(1, d), W2)
    a2 = agg(y2, srcp, dstp)
    out = final(a2[0, :n], a2[1, :n], y2, d0, d1, b2.reshape(1, d))
    return out
